# Initial kernel scaffold; baseline (speedup 1.0000x reference)
import jax
import jax.numpy as jnp
from jax.experimental import pallas as pl


def kernel(x, edges, W1, b1, Wa1, ba1, Wa2, ba2):
    raise NotImplementedError("write your pallas kernel here")


# speedup vs baseline: 18.7718x; 18.7718x over previous
import functools

import jax
import jax.numpy as jnp
from jax import lax
from jax.experimental import pallas as pl
from jax.experimental.pallas import tpu as pltpu
from jax.experimental.pallas import tpu_sc as plsc

N = 10000
FEATS = 128
CLASSES = 16
HIDDEN = 64
DEPTH = 10
DIFFUSION = 0.9

NTILES = 16
RPT = 632
NP = NTILES * RPT
CHUNK = 128
K = 160
EPT = K * CHUNK
EPAD = EPT * NTILES

_mesh = plsc.VectorSubcoreMesh(core_axis_name="c", subcore_axis_name="s")


def _deg_body(dst_hbm, deg_out, agg, dst_buf, ones_buf, zro_buf, dsem):
    core = lax.axis_index("c")
    t = lax.axis_index("s")
    rows = pl.ds(t * RPT, RPT)

    @pl.when(core == 0)
    def _():
        pltpu.sync_copy(dst_hbm.at[t], dst_buf)
        one = jnp.full((16,), 1.0, jnp.float32)
        zer = jnp.zeros((16,), jnp.float32)

        @pl.loop(0, CHUNK)
        def _(r):
            ones_buf[r, :] = one

        @pl.loop(0, RPT)
        def _(r):
            zro_buf[r, :] = zer

        pltpu.sync_copy(zro_buf, agg.at[rows])

    plsc.subcore_barrier()

    @pl.when(core == 0)
    def _():
        @pl.loop(0, K, step=4)
        def _(j):
            for q in range(4):
                pltpu.async_copy(ones_buf, agg.at[dst_buf.at[j + q]],
                                 dsem, add=True)
            for q in range(4):
                pltpu.make_async_copy(ones_buf,
                                      agg.at[dst_buf.at[j + q]],
                                      dsem).wait()

    plsc.subcore_barrier()

    @pl.when(core == 0)
    def _():
        pltpu.sync_copy(agg.at[rows], deg_out.at[rows])


def _deg_kernel(dst_tiled):
    return pl.kernel(
        _deg_body,
        out_type=jax.ShapeDtypeStruct((NP, CLASSES), jnp.float32),
        mesh=_mesh,
        compiler_params=pltpu.CompilerParams(use_tc_tiling_on_sc=False),
        scratch_types=[
            pltpu.MemorySpace.VMEM_SHARED((NP, CLASSES), jnp.float32),
            pltpu.VMEM((K, CHUNK), jnp.int32),
            pltpu.VMEM((CHUNK, CLASSES), jnp.float32),
            pltpu.VMEM((RPT, CLASSES), jnp.float32),
            pltpu.SemaphoreType.DMA,
        ],
    )(dst_tiled)


def _diff_body(src_hbm, dst_hbm, u0_hbm, u0t_hbm, c1_hbm, osc_hbm,
               out_hbm, agg,
               src_buf, dst_buf, msga, msgb, u_sl, t0_sl, c1_sl, agg_sl,
               zro_sl, osc_sl, gsa, gsb, ssa, ssb):
    core = lax.axis_index("c")
    t = lax.axis_index("s")
    rows = pl.ds(t * RPT, RPT)

    @pl.when(core == 0)
    def _():
        pltpu.sync_copy(src_hbm.at[t], src_buf)
        pltpu.sync_copy(dst_hbm.at[t], dst_buf)
        pltpu.sync_copy(u0_hbm.at[rows], u_sl)
        pltpu.sync_copy(u0t_hbm.at[rows], t0_sl)
        pltpu.sync_copy(c1_hbm.at[rows], c1_sl)
        pltpu.sync_copy(osc_hbm.at[rows], osc_sl)
        zer = jnp.zeros((16,), jnp.float32)

        @pl.loop(0, RPT)
        def _(r):
            zro_sl[r, :] = zer

        pltpu.sync_copy(zro_sl, agg.at[rows])
        pltpu.sync_copy(u_sl, out_hbm.at[rows])

    plsc.subcore_barrier()

    @pl.loop(0, DEPTH)
    def _(it):
        @pl.when(core == 0)
        def _():
            pltpu.async_copy(out_hbm.at[src_buf.at[0]], msga, gsa)
            pltpu.async_copy(out_hbm.at[src_buf.at[1]], msgb, gsb)

            @pl.loop(0, K, step=2)
            def _(j):
                pltpu.make_async_copy(out_hbm.at[src_buf.at[j]],
                                      msga, gsa).wait()
                pltpu.async_copy(msga, agg.at[dst_buf.at[j]], ssa, add=True)
                pltpu.make_async_copy(out_hbm.at[src_buf.at[j + 1]],
                                      msgb, gsb).wait()
                pltpu.async_copy(msgb, agg.at[dst_buf.at[j + 1]], ssb,
                                 add=True)
                pltpu.make_async_copy(msga, agg.at[dst_buf.at[j]],
                                      ssa).wait()

                @pl.when(j + 2 < K)
                def _():
                    pltpu.async_copy(out_hbm.at[src_buf.at[j + 2]],
                                     msga, gsa)

                pltpu.make_async_copy(msgb, agg.at[dst_buf.at[j + 1]],
                                      ssb).wait()

                @pl.when(j + 3 < K)
                def _():
                    pltpu.async_copy(out_hbm.at[src_buf.at[j + 3]],
                                     msgb, gsb)

        plsc.subcore_barrier()

        @pl.when(core == 0)
        def _():
            pltpu.sync_copy(agg.at[rows], agg_sl)

            @pl.loop(0, RPT)
            def _(r):
                u_sl[r, :] = (c1_sl[r, :] * (agg_sl[r, :] + u_sl[r, :])
                              + t0_sl[r, :])

            pltpu.sync_copy(zro_sl, agg.at[rows])
            pltpu.sync_copy(u_sl, out_hbm.at[rows])

        plsc.subcore_barrier()

    @pl.when(core == 0)
    def _():
        @pl.loop(0, RPT)
        def _(r):
            u_sl[r, :] = u_sl[r, :] * osc_sl[r, :]

        pltpu.sync_copy(u_sl, out_hbm.at[rows])


def _diff_kernel(src_tiled, dst_tiled, u0, u0t, c1, osc):
    return pl.kernel(
        _diff_body,
        out_type=jax.ShapeDtypeStruct((NP, CLASSES), jnp.float32),
        mesh=_mesh,
        compiler_params=pltpu.CompilerParams(use_tc_tiling_on_sc=False),
        scratch_types=[
            pltpu.MemorySpace.VMEM_SHARED((NP, CLASSES), jnp.float32),
            pltpu.VMEM((K, CHUNK), jnp.int32),
            pltpu.VMEM((K, CHUNK), jnp.int32),
            pltpu.VMEM((CHUNK, CLASSES), jnp.float32),
            pltpu.VMEM((CHUNK, CLASSES), jnp.float32),
            pltpu.VMEM((RPT, CLASSES), jnp.float32),
            pltpu.VMEM((RPT, CLASSES), jnp.float32),
            pltpu.VMEM((RPT, CLASSES), jnp.float32),
            pltpu.VMEM((RPT, CLASSES), jnp.float32),
            pltpu.VMEM((RPT, CLASSES), jnp.float32),
            pltpu.VMEM((RPT, CLASSES), jnp.float32),
            pltpu.SemaphoreType.DMA,
            pltpu.SemaphoreType.DMA,
            pltpu.SemaphoreType.DMA,
            pltpu.SemaphoreType.DMA,
        ],
    )(src_tiled, dst_tiled, u0, u0t, c1, osc)


def _prep_body(deg_ref, x_ref, w1_ref, b1_ref, wf_ref, ba1_ref,
               u0_ref, u0t_ref, c1_ref, osc_ref, s_ref, pre_ref):
    deg = deg_ref[:, :] + 1.0
    s = lax.rsqrt(deg)
    h0 = lax.dot_general(x_ref[:, :], w1_ref[:, :],
                         (((1,), (1,)), ((), ())),
                         preferred_element_type=jnp.float32) + b1_ref[:, :]
    u0 = s * h0
    u0_ref[:, :] = u0
    u0t_ref[:, :] = 0.1 * u0
    c1_ref[:, :] = DIFFUSION * s * s
    osc_ref[:, :] = jnp.sqrt(deg)
    s_ref[:, :] = s
    pre_ref[:, :] = lax.dot_general(x_ref[:, :], wf_ref[:, :],
                                    (((1,), (1,)), ((), ())),
                                    preferred_element_type=jnp.float32
                                    ) + ba1_ref[:, :]


def _prep_kernel(deg, xp, W1, b1, Wfeat, ba1):
    blk = NP // 4
    o16 = jax.ShapeDtypeStruct((NP, CLASSES), jnp.float32)
    return pl.pallas_call(
        _prep_body,
        grid=(4,),
        in_specs=[
            pl.BlockSpec((blk, CLASSES), lambda i: (i, 0)),
            pl.BlockSpec((blk, FEATS), lambda i: (i, 0)),
            pl.BlockSpec((CLASSES, FEATS), lambda i: (0, 0)),
            pl.BlockSpec((1, CLASSES), lambda i: (0, 0)),
            pl.BlockSpec((HIDDEN, FEATS), lambda i: (0, 0)),
            pl.BlockSpec((1, HIDDEN), lambda i: (0, 0)),
        ],
        out_specs=[
            pl.BlockSpec((blk, CLASSES), lambda i: (i, 0)),
            pl.BlockSpec((blk, CLASSES), lambda i: (i, 0)),
            pl.BlockSpec((blk, CLASSES), lambda i: (i, 0)),
            pl.BlockSpec((blk, CLASSES), lambda i: (i, 0)),
            pl.BlockSpec((blk, CLASSES), lambda i: (i, 0)),
            pl.BlockSpec((blk, HIDDEN), lambda i: (i, 0)),
        ],
        out_shape=[o16, o16, o16, o16, o16,
                   jax.ShapeDtypeStruct((NP, HIDDEN), jnp.float32)],
    )(deg, xp, W1, b1, Wfeat, ba1)


def _mlp_body(h1_ref, pre_ref, wcol_ref, wind_ref, wa2_ref, ba2_ref, s_ref,
              u0_ref, u0t_ref):
    h1 = h1_ref[:, :]
    pre = pre_ref[:, :]
    wcol = wcol_ref[:, :]
    wa2 = wa2_ref[:, :]
    ba2 = ba2_ref[0, 0]
    cols = []
    for c in range(CLASSES):
        tz = jnp.maximum(pre + wind_ref[c:c + 1, :] + h1[:, c:c + 1] * wcol,
                         0.0)
        oc = lax.dot_general(tz, wa2, (((1,), (0,)), ((), ())),
                             preferred_element_type=jnp.float32)
        cols.append(oc)
    h2 = jnp.maximum(jnp.concatenate(cols, axis=1) + ba2, 0.0)
    u0 = s_ref[:, :] * h2
    u0_ref[:, :] = u0
    u0t_ref[:, :] = 0.1 * u0


def _mlp_kernel(h1, pre, wcol, Wind, wa2, ba2, s16):
    blk = NP // 4
    o16 = jax.ShapeDtypeStruct((NP, CLASSES), jnp.float32)
    return pl.pallas_call(
        _mlp_body,
        grid=(4,),
        in_specs=[
            pl.BlockSpec((blk, CLASSES), lambda i: (i, 0)),
            pl.BlockSpec((blk, HIDDEN), lambda i: (i, 0)),
            pl.BlockSpec((1, HIDDEN), lambda i: (0, 0)),
            pl.BlockSpec((CLASSES, HIDDEN), lambda i: (0, 0)),
            pl.BlockSpec((HIDDEN, 1), lambda i: (0, 0)),
            pl.BlockSpec((1, 1), lambda i: (0, 0)),
            pl.BlockSpec((blk, CLASSES), lambda i: (i, 0)),
        ],
        out_specs=[
            pl.BlockSpec((blk, CLASSES), lambda i: (i, 0)),
            pl.BlockSpec((blk, CLASSES), lambda i: (i, 0)),
        ],
        out_shape=[o16, o16],
    )(h1, pre, wcol, Wind, wa2, ba2, s16)


@jax.jit
def kernel(x, edges, W1, b1, Wa1, ba1, Wa2, ba2):
    src, dst = edges[0], edges[1]
    npad = EPAD - src.shape[0]
    pad = (N + (jnp.arange(npad, dtype=jnp.int32) % (NP - N))).astype(jnp.int32)
    src_t = jnp.concatenate([src, pad]).reshape(NTILES, K, CHUNK)
    dst_t = jnp.concatenate([dst, pad]).reshape(NTILES, K, CHUNK)
    xp = jnp.pad(x, ((0, NP - N), (0, 0)))

    wcol = Wa1[:, 0:1].T
    Wind = Wa1[:, 1:1 + CLASSES].T
    Wfeat = Wa1[:, 1 + CLASSES:]

    deg = _deg_kernel(dst_t)
    u0, u0t, c1, osc, s16, pre = _prep_kernel(
        deg, xp, W1, b1.reshape(1, CLASSES), Wfeat, ba1.reshape(1, HIDDEN))
    h1 = _diff_kernel(src_t, dst_t, u0, u0t, c1, osc)
    u0b, u0bt = _mlp_kernel(h1, pre, wcol, Wind, Wa2.T, ba2.reshape(1, 1),
                            s16)
    h2 = _diff_kernel(src_t, dst_t, u0b, u0bt, c1, osc)
    return h2[:N]


# speedup vs baseline: 30.0625x; 1.6015x over previous
import jax
import jax.numpy as jnp
from jax import lax
from jax.experimental import pallas as pl
from jax.experimental.pallas import tpu as pltpu
from jax.experimental.pallas import tpu_sc as plsc

N = 10000
FEATS = 128
CLASSES = 16
HIDDEN = 64
DEPTH = 10
DIFFUSION = 0.9

NTILES = 16
NCORES = 2
RPT = 640
NP = NTILES * RPT
CHUNK = 128
K = 80
EPT = K * CHUNK
EPAD = EPT * NTILES * NCORES

_mesh = plsc.VectorSubcoreMesh(core_axis_name="c", subcore_axis_name="s")


def _deg_body(dst_hbm, deg_out, agg, dst_buf, ones_buf, mine_buf, other_buf,
              dsem, xsem):
    core = lax.axis_index("c")
    t = lax.axis_index("s")
    rows = pl.ds(t * RPT, RPT)
    widx = core * NTILES + t

    pltpu.sync_copy(dst_hbm.at[widx], dst_buf)
    one = jnp.full((16,), 1.0, jnp.float32)
    zer = jnp.zeros((16,), jnp.float32)

    @pl.loop(0, CHUNK)
    def _(r):
        ones_buf[r, :] = one

    @pl.loop(0, RPT)
    def _(r):
        mine_buf[r, :] = zer

    pltpu.sync_copy(mine_buf, agg.at[rows])
    plsc.subcore_barrier()

    @pl.loop(0, K, step=4)
    def _(j):
        for q in range(4):
            pltpu.async_copy(ones_buf, agg.at[dst_buf.at[j + q]],
                             dsem, add=True)
        for q in range(4):
            pltpu.make_async_copy(ones_buf,
                                  agg.at[dst_buf.at[j + q]],
                                  dsem).wait()

    plsc.subcore_barrier()
    @pl.when(core == 1)
    def _():
        pltpu.sync_copy(agg.at[rows], deg_out.at[rows])

    pl.semaphore_signal(xsem, 1, core_index=1 - core)
    pl.semaphore_wait(xsem, 1)

    @pl.when(core == 0)
    def _():
        pltpu.sync_copy(deg_out.at[rows], other_buf)
        pltpu.sync_copy(agg.at[rows], mine_buf)

        @pl.loop(0, RPT)
        def _(r):
            mine_buf[r, :] = mine_buf[r, :] + other_buf[r, :]

        pltpu.sync_copy(mine_buf, deg_out.at[rows])


def _deg_kernel(dst_tiled):
    return pl.kernel(
        _deg_body,
        out_type=jax.ShapeDtypeStruct((NP, CLASSES), jnp.float32),
        mesh=_mesh,
        compiler_params=pltpu.CompilerParams(use_tc_tiling_on_sc=False),
        scratch_types=[
            pltpu.MemorySpace.VMEM_SHARED((NP, CLASSES), jnp.float32),
            pltpu.VMEM((K, CHUNK), jnp.int32),
            pltpu.VMEM((CHUNK, CLASSES), jnp.float32),
            pltpu.VMEM((RPT, CLASSES), jnp.float32),
            pltpu.VMEM((RPT, CLASSES), jnp.float32),
            pltpu.SemaphoreType.DMA,
            pltpu.SemaphoreType.REGULAR,
        ],
    )(dst_tiled)


def _diff_body(src_hbm, dst_hbm, u0_hbm, u0t_hbm, c1_hbm, osc_hbm,
               out_hbm, utab0, utab1, part0, part1, agg,
               src_buf, dst_buf, msga, msgb, u_sl, t0_sl, c1_sl, agg_sl,
               aggo_sl, zro_sl, gsa, gsb, ssa, ssb, xsem):
    core = lax.axis_index("c")
    t = lax.axis_index("s")
    rows = pl.ds(t * RPT, RPT)
    widx = core * NTILES + t

    pltpu.sync_copy(src_hbm.at[widx], src_buf)
    pltpu.sync_copy(dst_hbm.at[widx], dst_buf)
    pltpu.sync_copy(u0_hbm.at[rows], u_sl)
    pltpu.sync_copy(u0t_hbm.at[rows], t0_sl)
    pltpu.sync_copy(c1_hbm.at[rows], c1_sl)
    zer = jnp.zeros((16,), jnp.float32)

    @pl.loop(0, RPT)
    def _(r):
        zro_sl[r, :] = zer

    pltpu.sync_copy(zro_sl, agg.at[rows])

    @pl.when(core == 0)
    def _():
        pltpu.sync_copy(u_sl, utab0.at[rows])

    @pl.when(core == 1)
    def _():
        pltpu.sync_copy(u_sl, utab1.at[rows])

    plsc.subcore_barrier()

    def scatter_phase(utab):
        pltpu.async_copy(utab.at[src_buf.at[0]], msga, gsa)
        pltpu.async_copy(utab.at[src_buf.at[1]], msgb, gsb)

        @pl.loop(0, K, step=2)
        def _(j):
            pltpu.make_async_copy(utab.at[src_buf.at[j]], msga, gsa).wait()
            pltpu.async_copy(msga, agg.at[dst_buf.at[j]], ssa, add=True)
            pltpu.make_async_copy(utab.at[src_buf.at[j + 1]], msgb,
                                  gsb).wait()
            pltpu.async_copy(msgb, agg.at[dst_buf.at[j + 1]], ssb, add=True)
            pltpu.make_async_copy(msga, agg.at[dst_buf.at[j]], ssa).wait()

            @pl.when(j + 2 < K)
            def _():
                pltpu.async_copy(utab.at[src_buf.at[j + 2]], msga, gsa)

            pltpu.make_async_copy(msgb, agg.at[dst_buf.at[j + 1]],
                                  ssb).wait()

            @pl.when(j + 3 < K)
            def _():
                pltpu.async_copy(utab.at[src_buf.at[j + 3]], msgb, gsb)

    def publish_and_update(mypart, otherpart, myutab):
        pltpu.sync_copy(agg.at[rows], mypart.at[rows])
        pl.semaphore_signal(xsem, 1, core_index=1 - core)
        pl.semaphore_wait(xsem, 1)
        pltpu.sync_copy(agg.at[rows], agg_sl)
        pltpu.sync_copy(otherpart.at[rows], aggo_sl)

        @pl.loop(0, RPT)
        def _(r):
            u_sl[r, :] = (c1_sl[r, :]
                          * (agg_sl[r, :] + aggo_sl[r, :] + u_sl[r, :])
                          + t0_sl[r, :])

        pltpu.sync_copy(zro_sl, agg.at[rows])
        pltpu.sync_copy(u_sl, myutab.at[rows])

    @pl.loop(0, DEPTH)
    def _(it):
        @pl.when(core == 0)
        def _():
            scatter_phase(utab0)

        @pl.when(core == 1)
        def _():
            scatter_phase(utab1)

        plsc.subcore_barrier()

        @pl.when(core == 0)
        def _():
            publish_and_update(part0, part1, utab0)

        @pl.when(core == 1)
        def _():
            publish_and_update(part1, part0, utab1)

        plsc.subcore_barrier()

    @pl.when(core == 0)
    def _():
        pltpu.sync_copy(osc_hbm.at[rows], t0_sl)

        @pl.loop(0, RPT)
        def _(r):
            u_sl[r, :] = u_sl[r, :] * t0_sl[r, :]

        pltpu.sync_copy(u_sl, out_hbm.at[rows])


def _diff_kernel(src_tiled, dst_tiled, u0, u0t, c1, osc):
    o16 = jax.ShapeDtypeStruct((NP, CLASSES), jnp.float32)
    outs = pl.kernel(
        _diff_body,
        out_type=[o16, o16, o16, o16, o16],
        mesh=_mesh,
        compiler_params=pltpu.CompilerParams(use_tc_tiling_on_sc=False),
        scratch_types=[
            pltpu.MemorySpace.VMEM_SHARED((NP, CLASSES), jnp.float32),
            pltpu.VMEM((K, CHUNK), jnp.int32),
            pltpu.VMEM((K, CHUNK), jnp.int32),
            pltpu.VMEM((CHUNK, CLASSES), jnp.float32),
            pltpu.VMEM((CHUNK, CLASSES), jnp.float32),
            pltpu.VMEM((RPT, CLASSES), jnp.float32),
            pltpu.VMEM((RPT, CLASSES), jnp.float32),
            pltpu.VMEM((RPT, CLASSES), jnp.float32),
            pltpu.VMEM((RPT, CLASSES), jnp.float32),
            pltpu.VMEM((RPT, CLASSES), jnp.float32),
            pltpu.VMEM((RPT, CLASSES), jnp.float32),
            pltpu.SemaphoreType.DMA,
            pltpu.SemaphoreType.DMA,
            pltpu.SemaphoreType.DMA,
            pltpu.SemaphoreType.DMA,
            pltpu.SemaphoreType.REGULAR,
        ],
    )(src_tiled, dst_tiled, u0, u0t, c1, osc)
    return outs[0]


def _prep_body(deg_ref, x_ref, w1_ref, b1_ref, wf_ref, ba1_ref,
               u0_ref, u0t_ref, c1_ref, osc_ref, s_ref, pre_ref):
    deg = deg_ref[:, :] + 1.0
    s = lax.rsqrt(deg)
    h0 = lax.dot_general(x_ref[:, :], w1_ref[:, :],
                         (((1,), (1,)), ((), ())),
                         preferred_element_type=jnp.float32) + b1_ref[:, :]
    u0 = s * h0
    u0_ref[:, :] = u0
    u0t_ref[:, :] = 0.1 * u0
    c1_ref[:, :] = DIFFUSION * s * s
    osc_ref[:, :] = jnp.sqrt(deg)
    s_ref[:, :] = s
    pre_ref[:, :] = lax.dot_general(x_ref[:, :], wf_ref[:, :],
                                    (((1,), (1,)), ((), ())),
                                    preferred_element_type=jnp.float32
                                    ) + ba1_ref[:, :]


def _prep_kernel(deg, xp, W1, b1, Wfeat, ba1):
    blk = NP // 4
    o16 = jax.ShapeDtypeStruct((NP, CLASSES), jnp.float32)
    return pl.pallas_call(
        _prep_body,
        grid=(4,),
        in_specs=[
            pl.BlockSpec((blk, CLASSES), lambda i: (i, 0)),
            pl.BlockSpec((blk, FEATS), lambda i: (i, 0)),
            pl.BlockSpec((CLASSES, FEATS), lambda i: (0, 0)),
            pl.BlockSpec((1, CLASSES), lambda i: (0, 0)),
            pl.BlockSpec((HIDDEN, FEATS), lambda i: (0, 0)),
            pl.BlockSpec((1, HIDDEN), lambda i: (0, 0)),
        ],
        out_specs=[
            pl.BlockSpec((blk, CLASSES), lambda i: (i, 0)),
            pl.BlockSpec((blk, CLASSES), lambda i: (i, 0)),
            pl.BlockSpec((blk, CLASSES), lambda i: (i, 0)),
            pl.BlockSpec((blk, CLASSES), lambda i: (i, 0)),
            pl.BlockSpec((blk, CLASSES), lambda i: (i, 0)),
            pl.BlockSpec((blk, HIDDEN), lambda i: (i, 0)),
        ],
        out_shape=[o16, o16, o16, o16, o16,
                   jax.ShapeDtypeStruct((NP, HIDDEN), jnp.float32)],
    )(deg, xp, W1, b1, Wfeat, ba1)


def _mlp_body(h1_ref, pre_ref, wcol_ref, wind_ref, wa2_ref, ba2_ref, s_ref,
              u0_ref, u0t_ref):
    h1 = h1_ref[:, :]
    pre = pre_ref[:, :]
    wcol = wcol_ref[:, :]
    wa2 = wa2_ref[:, :]
    ba2 = ba2_ref[0, 0]
    cols = []
    for c in range(CLASSES):
        tz = jnp.maximum(pre + wind_ref[c:c + 1, :] + h1[:, c:c + 1] * wcol,
                         0.0)
        oc = lax.dot_general(tz, wa2, (((1,), (0,)), ((), ())),
                             preferred_element_type=jnp.float32)
        cols.append(oc)
    h2 = jnp.maximum(jnp.concatenate(cols, axis=1) + ba2, 0.0)
    u0 = s_ref[:, :] * h2
    u0_ref[:, :] = u0
    u0t_ref[:, :] = 0.1 * u0


def _mlp_kernel(h1, pre, wcol, Wind, wa2, ba2, s16):
    blk = NP // 4
    o16 = jax.ShapeDtypeStruct((NP, CLASSES), jnp.float32)
    return pl.pallas_call(
        _mlp_body,
        grid=(4,),
        in_specs=[
            pl.BlockSpec((blk, CLASSES), lambda i: (i, 0)),
            pl.BlockSpec((blk, HIDDEN), lambda i: (i, 0)),
            pl.BlockSpec((1, HIDDEN), lambda i: (0, 0)),
            pl.BlockSpec((CLASSES, HIDDEN), lambda i: (0, 0)),
            pl.BlockSpec((HIDDEN, 1), lambda i: (0, 0)),
            pl.BlockSpec((1, 1), lambda i: (0, 0)),
            pl.BlockSpec((blk, CLASSES), lambda i: (i, 0)),
        ],
        out_specs=[
            pl.BlockSpec((blk, CLASSES), lambda i: (i, 0)),
            pl.BlockSpec((blk, CLASSES), lambda i: (i, 0)),
        ],
        out_shape=[o16, o16],
    )(h1, pre, wcol, Wind, wa2, ba2, s16)


@jax.jit
def kernel(x, edges, W1, b1, Wa1, ba1, Wa2, ba2):
    src, dst = edges[0], edges[1]
    npad = EPAD - src.shape[0]
    pad = (N + (jnp.arange(npad, dtype=jnp.int32) % (NP - N))).astype(jnp.int32)
    src_t = jnp.concatenate([src, pad]).reshape(NTILES * NCORES, K, CHUNK)
    dst_t = jnp.concatenate([dst, pad]).reshape(NTILES * NCORES, K, CHUNK)
    xp = jnp.pad(x, ((0, NP - N), (0, 0)))

    wcol = Wa1[:, 0:1].T
    Wind = Wa1[:, 1:1 + CLASSES].T
    Wfeat = Wa1[:, 1 + CLASSES:]

    deg = _deg_kernel(dst_t)
    u0, u0t, c1, osc, s16, pre = _prep_kernel(
        deg, xp, W1, b1.reshape(1, CLASSES), Wfeat, ba1.reshape(1, HIDDEN))
    h1 = _diff_kernel(src_t, dst_t, u0, u0t, c1, osc)
    u0b, u0bt = _mlp_kernel(h1, pre, wcol, Wind, Wa2.T, ba2.reshape(1, 1),
                            s16)
    h2 = _diff_kernel(src_t, dst_t, u0b, u0bt, c1, osc)
    return h2[:N]


# speedup vs baseline: 40.6442x; 1.3520x over previous
import jax
import jax.numpy as jnp
from jax import lax
from jax.experimental import pallas as pl
from jax.experimental.pallas import tpu as pltpu
from jax.experimental.pallas import tpu_sc as plsc

N = 10000
FEATS = 128
CLASSES = 16
HIDDEN = 64
DEPTH = 10
DIFFUSION = 0.9

NTILES = 16
NCORES = 2
RPT = 640
NP = NTILES * RPT
CHUNK = 128
K = 80
EPT = K * CHUNK
EPAD = EPT * NTILES * NCORES

_mesh = plsc.VectorSubcoreMesh(core_axis_name="c", subcore_axis_name="s")


def _deg_body(dst_hbm, deg_out, agg, dst_buf, ones_buf, mine_buf, other_buf,
              dsem, xsem):
    core = lax.axis_index("c")
    t = lax.axis_index("s")
    rows = pl.ds(t * RPT, RPT)
    widx = core * NTILES + t

    pltpu.sync_copy(dst_hbm.at[widx], dst_buf)
    one = jnp.full((16,), 1.0, jnp.float32)
    zer = jnp.zeros((16,), jnp.float32)

    @pl.loop(0, CHUNK)
    def _(r):
        ones_buf[r, :] = one

    @pl.loop(0, RPT)
    def _(r):
        mine_buf[r, :] = zer

    pltpu.sync_copy(mine_buf, agg.at[rows])
    plsc.subcore_barrier()

    @pl.loop(0, K, step=4)
    def _(j):
        for q in range(4):
            pltpu.async_copy(ones_buf, agg.at[dst_buf.at[j + q]],
                             dsem, add=True)
        for q in range(4):
            pltpu.make_async_copy(ones_buf,
                                  agg.at[dst_buf.at[j + q]],
                                  dsem).wait()

    plsc.subcore_barrier()
    @pl.when(core == 1)
    def _():
        pltpu.sync_copy(agg.at[rows], deg_out.at[rows])

    pl.semaphore_signal(xsem, 1, core_index=1 - core)
    pl.semaphore_wait(xsem, 1)

    @pl.when(core == 0)
    def _():
        pltpu.sync_copy(deg_out.at[rows], other_buf)
        pltpu.sync_copy(agg.at[rows], mine_buf)

        @pl.loop(0, RPT)
        def _(r):
            mine_buf[r, :] = mine_buf[r, :] + other_buf[r, :]

        pltpu.sync_copy(mine_buf, deg_out.at[rows])


def _deg_kernel(dst_tiled):
    return pl.kernel(
        _deg_body,
        out_type=jax.ShapeDtypeStruct((NP, CLASSES), jnp.float32),
        mesh=_mesh,
        compiler_params=pltpu.CompilerParams(use_tc_tiling_on_sc=False),
        scratch_types=[
            pltpu.MemorySpace.VMEM_SHARED((NP, CLASSES), jnp.float32),
            pltpu.VMEM((K, CHUNK), jnp.int32),
            pltpu.VMEM((CHUNK, CLASSES), jnp.float32),
            pltpu.VMEM((RPT, CLASSES), jnp.float32),
            pltpu.VMEM((RPT, CLASSES), jnp.float32),
            pltpu.SemaphoreType.DMA,
            pltpu.SemaphoreType.REGULAR,
        ],
    )(dst_tiled)


def _diff_body(src_hbm, dst_hbm, u0_hbm, u0t_hbm, c1_hbm, osc_hbm,
               out_hbm, utab0, utab1, part0, part1, agg,
               src_buf, dst_buf, msga, msgb, msgc, msgd, u_sl, t0_sl, c1_sl,
               agg_sl, aggo_sl, zro_sl, gsa, gsb, gsc, gsd, ssa, ssb, ssc,
               ssd, xsem):
    core = lax.axis_index("c")
    t = lax.axis_index("s")
    rows = pl.ds(t * RPT, RPT)
    widx = core * NTILES + t

    pltpu.sync_copy(src_hbm.at[widx], src_buf)
    pltpu.sync_copy(dst_hbm.at[widx], dst_buf)
    pltpu.sync_copy(u0_hbm.at[rows], u_sl)
    pltpu.sync_copy(u0t_hbm.at[rows], t0_sl)
    pltpu.sync_copy(c1_hbm.at[rows], c1_sl)
    zer = jnp.zeros((16,), jnp.float32)

    @pl.loop(0, RPT)
    def _(r):
        zro_sl[r, :] = zer

    pltpu.sync_copy(zro_sl, agg.at[rows])

    @pl.when(core == 0)
    def _():
        pltpu.sync_copy(u_sl, utab0.at[rows])

    @pl.when(core == 1)
    def _():
        pltpu.sync_copy(u_sl, utab1.at[rows])

    plsc.subcore_barrier()

    msgs = (msga, msgb, msgc, msgd)
    gsems = (gsa, gsb, gsc, gsd)
    ssems = (ssa, ssb, ssc, ssd)

    def scatter_phase(utab):
        for q in range(4):
            pltpu.async_copy(utab.at[src_buf.at[q]], msgs[q], gsems[q])

        @pl.loop(0, K, step=4)
        def _(j):
            for q in range(4):
                pltpu.make_async_copy(utab.at[src_buf.at[j + q]],
                                      msgs[q], gsems[q]).wait()
                pltpu.async_copy(msgs[q], agg.at[dst_buf.at[j + q]],
                                 ssems[q], add=True)
            for q in range(4):
                pltpu.make_async_copy(msgs[q], agg.at[dst_buf.at[j + q]],
                                      ssems[q]).wait()

                @pl.when(j + q + 4 < K)
                def _():
                    pltpu.async_copy(utab.at[src_buf.at[j + q + 4]],
                                     msgs[q], gsems[q])

    def publish_and_update(mypart, otherpart, myutab):
        pltpu.sync_copy(agg.at[rows], mypart.at[rows])
        pl.semaphore_signal(xsem, 1, core_index=1 - core)
        pl.semaphore_wait(xsem, 1)
        pltpu.sync_copy(agg.at[rows], agg_sl)
        pltpu.sync_copy(otherpart.at[rows], aggo_sl)

        @pl.loop(0, RPT, unroll=4)
        def _(r):
            u_sl[r, :] = (c1_sl[r, :]
                          * (agg_sl[r, :] + aggo_sl[r, :] + u_sl[r, :])
                          + t0_sl[r, :])

        pltpu.sync_copy(zro_sl, agg.at[rows])
        pltpu.sync_copy(u_sl, myutab.at[rows])

    @pl.loop(0, DEPTH)
    def _(it):
        @pl.when(core == 0)
        def _():
            scatter_phase(utab0)

        @pl.when(core == 1)
        def _():
            scatter_phase(utab1)

        plsc.subcore_barrier()

        @pl.when(core == 0)
        def _():
            publish_and_update(part0, part1, utab0)

        @pl.when(core == 1)
        def _():
            publish_and_update(part1, part0, utab1)

        plsc.subcore_barrier()

    @pl.when(core == 0)
    def _():
        pltpu.sync_copy(osc_hbm.at[rows], t0_sl)

        @pl.loop(0, RPT)
        def _(r):
            u_sl[r, :] = u_sl[r, :] * t0_sl[r, :]

        pltpu.sync_copy(u_sl, out_hbm.at[rows])


def _diff_kernel(src_tiled, dst_tiled, u0, u0t, c1, osc):
    o16 = jax.ShapeDtypeStruct((NP, CLASSES), jnp.float32)
    outs = pl.kernel(
        _diff_body,
        out_type=[o16, o16, o16, o16, o16],
        mesh=_mesh,
        compiler_params=pltpu.CompilerParams(use_tc_tiling_on_sc=False),
        scratch_types=[
            pltpu.MemorySpace.VMEM_SHARED((NP, CLASSES), jnp.float32),
            pltpu.VMEM((K, CHUNK), jnp.int32),
            pltpu.VMEM((K, CHUNK), jnp.int32),
            pltpu.VMEM((CHUNK, CLASSES), jnp.float32),
            pltpu.VMEM((CHUNK, CLASSES), jnp.float32),
            pltpu.VMEM((CHUNK, CLASSES), jnp.float32),
            pltpu.VMEM((CHUNK, CLASSES), jnp.float32),
            pltpu.VMEM((RPT, CLASSES), jnp.float32),
            pltpu.VMEM((RPT, CLASSES), jnp.float32),
            pltpu.VMEM((RPT, CLASSES), jnp.float32),
            pltpu.VMEM((RPT, CLASSES), jnp.float32),
            pltpu.VMEM((RPT, CLASSES), jnp.float32),
            pltpu.VMEM((RPT, CLASSES), jnp.float32),
            pltpu.SemaphoreType.DMA,
            pltpu.SemaphoreType.DMA,
            pltpu.SemaphoreType.DMA,
            pltpu.SemaphoreType.DMA,
            pltpu.SemaphoreType.DMA,
            pltpu.SemaphoreType.DMA,
            pltpu.SemaphoreType.DMA,
            pltpu.SemaphoreType.DMA,
            pltpu.SemaphoreType.REGULAR,
        ],
    )(src_tiled, dst_tiled, u0, u0t, c1, osc)
    return outs[0]


def _prep_body(deg_ref, x_ref, w1_ref, b1_ref, wf_ref, ba1_ref,
               u0_ref, u0t_ref, c1_ref, osc_ref, s_ref, pre_ref):
    deg = deg_ref[:, :] + 1.0
    s = lax.rsqrt(deg)
    h0 = lax.dot_general(x_ref[:, :], w1_ref[:, :],
                         (((1,), (1,)), ((), ())),
                         preferred_element_type=jnp.float32) + b1_ref[:, :]
    u0 = s * h0
    u0_ref[:, :] = u0
    u0t_ref[:, :] = 0.1 * u0
    c1_ref[:, :] = DIFFUSION * s * s
    osc_ref[:, :] = jnp.sqrt(deg)
    s_ref[:, :] = s
    pre_ref[:, :] = lax.dot_general(x_ref[:, :], wf_ref[:, :],
                                    (((1,), (1,)), ((), ())),
                                    preferred_element_type=jnp.float32
                                    ) + ba1_ref[:, :]


def _prep_kernel(deg, xp, W1, b1, Wfeat, ba1):
    blk = NP // 4
    o16 = jax.ShapeDtypeStruct((NP, CLASSES), jnp.float32)
    return pl.pallas_call(
        _prep_body,
        grid=(4,),
        in_specs=[
            pl.BlockSpec((blk, CLASSES), lambda i: (i, 0)),
            pl.BlockSpec((blk, FEATS), lambda i: (i, 0)),
            pl.BlockSpec((CLASSES, FEATS), lambda i: (0, 0)),
            pl.BlockSpec((1, CLASSES), lambda i: (0, 0)),
            pl.BlockSpec((HIDDEN, FEATS), lambda i: (0, 0)),
            pl.BlockSpec((1, HIDDEN), lambda i: (0, 0)),
        ],
        out_specs=[
            pl.BlockSpec((blk, CLASSES), lambda i: (i, 0)),
            pl.BlockSpec((blk, CLASSES), lambda i: (i, 0)),
            pl.BlockSpec((blk, CLASSES), lambda i: (i, 0)),
            pl.BlockSpec((blk, CLASSES), lambda i: (i, 0)),
            pl.BlockSpec((blk, CLASSES), lambda i: (i, 0)),
            pl.BlockSpec((blk, HIDDEN), lambda i: (i, 0)),
        ],
        out_shape=[o16, o16, o16, o16, o16,
                   jax.ShapeDtypeStruct((NP, HIDDEN), jnp.float32)],
    )(deg, xp, W1, b1, Wfeat, ba1)


def _mlp_body(h1_ref, pre_ref, wcol_ref, wind_ref, wa2_ref, ba2_ref, s_ref,
              u0_ref, u0t_ref):
    h1 = h1_ref[:, :]
    pre = pre_ref[:, :]
    wcol = wcol_ref[:, :]
    wa2 = wa2_ref[:, :]
    ba2 = ba2_ref[0, 0]
    cols = []
    for c in range(CLASSES):
        tz = jnp.maximum(pre + wind_ref[c:c + 1, :] + h1[:, c:c + 1] * wcol,
                         0.0)
        oc = lax.dot_general(tz, wa2, (((1,), (0,)), ((), ())),
                             preferred_element_type=jnp.float32)
        cols.append(oc)
    h2 = jnp.maximum(jnp.concatenate(cols, axis=1) + ba2, 0.0)
    u0 = s_ref[:, :] * h2
    u0_ref[:, :] = u0
    u0t_ref[:, :] = 0.1 * u0


def _mlp_kernel(h1, pre, wcol, Wind, wa2, ba2, s16):
    blk = NP // 4
    o16 = jax.ShapeDtypeStruct((NP, CLASSES), jnp.float32)
    return pl.pallas_call(
        _mlp_body,
        grid=(4,),
        in_specs=[
            pl.BlockSpec((blk, CLASSES), lambda i: (i, 0)),
            pl.BlockSpec((blk, HIDDEN), lambda i: (i, 0)),
            pl.BlockSpec((1, HIDDEN), lambda i: (0, 0)),
            pl.BlockSpec((CLASSES, HIDDEN), lambda i: (0, 0)),
            pl.BlockSpec((HIDDEN, 1), lambda i: (0, 0)),
            pl.BlockSpec((1, 1), lambda i: (0, 0)),
            pl.BlockSpec((blk, CLASSES), lambda i: (i, 0)),
        ],
        out_specs=[
            pl.BlockSpec((blk, CLASSES), lambda i: (i, 0)),
            pl.BlockSpec((blk, CLASSES), lambda i: (i, 0)),
        ],
        out_shape=[o16, o16],
    )(h1, pre, wcol, Wind, wa2, ba2, s16)


@jax.jit
def kernel(x, edges, W1, b1, Wa1, ba1, Wa2, ba2):
    src, dst = edges[0], edges[1]
    npad = EPAD - src.shape[0]
    pad = (N + (jnp.arange(npad, dtype=jnp.int32) % (NP - N))).astype(jnp.int32)
    src_t = jnp.concatenate([src, pad]).reshape(NTILES * NCORES, K, CHUNK)
    dst_t = jnp.concatenate([dst, pad]).reshape(NTILES * NCORES, K, CHUNK)
    xp = jnp.pad(x, ((0, NP - N), (0, 0)))

    wcol = Wa1[:, 0:1].T
    Wind = Wa1[:, 1:1 + CLASSES].T
    Wfeat = Wa1[:, 1 + CLASSES:]

    deg = _deg_kernel(dst_t)
    u0, u0t, c1, osc, s16, pre = _prep_kernel(
        deg, xp, W1, b1.reshape(1, CLASSES), Wfeat, ba1.reshape(1, HIDDEN))
    h1 = _diff_kernel(src_t, dst_t, u0, u0t, c1, osc)
    u0b, u0bt = _mlp_kernel(h1, pre, wcol, Wind, Wa2.T, ba2.reshape(1, 1),
                            s16)
    h2 = _diff_kernel(src_t, dst_t, u0b, u0bt, c1, osc)
    return h2[:N]
